# baseline (speedup 1.0000x reference)
import jax
import jax.numpy as jnp
from jax.experimental import pallas as pl
from jax.experimental.pallas import tpu as pltpu

N = 4096
BM = 512
NB = N // BM


def _prep_kernel(x_ref, w1_ref, scale_ref, t_ref):
    t = jnp.dot(x_ref[...], w1_ref[...], preferred_element_type=jnp.float32)
    t_ref[...] = (t * scale_ref[...]).astype(jnp.bfloat16)


def _gcn_kernel(adj_ref, t_ref, c_ref, wa_ref, ba_ref,
                o_ref, adjv_ref, p_ref, acc_ref):
    j = pl.program_id(0)

    @pl.when(j == 0)
    def _zero():
        p_ref[...] = jnp.zeros((N, p_ref.shape[1]), jnp.bfloat16)

    ablk = adj_ref[...].astype(jnp.bfloat16)
    for k in range(NB):
        adjv_ref[k, pl.ds(j * BM, BM), :] = ablk[:, k * BM:(k + 1) * BM]

    h = jnp.dot(ablk, t_ref[...], preferred_element_type=jnp.float32)
    h = jnp.maximum(h + c_ref[...], 0.0)
    pj = jnp.dot(h.astype(jnp.bfloat16), wa_ref[...],
                 preferred_element_type=jnp.float32).astype(jnp.bfloat16)

    acc_ref[pl.ds(j * BM, BM), :] = jnp.dot(ablk, p_ref[...],
                                            preferred_element_type=jnp.float32)
    p_ref[pl.ds(j * BM, BM), :] = pj

    for i in range(NB):
        @pl.when(j >= i)
        def _caseb(i=i):
            blk = jnp.dot(adjv_ref[j, i * BM:(i + 1) * BM, :], pj,
                          preferred_element_type=jnp.float32)
            acc_ref[i * BM:(i + 1) * BM, :] += blk

    @pl.when(j == NB - 1)
    def _final():
        for i in range(NB):
            logits = acc_ref[i * BM:(i + 1) * BM, :] + ba_ref[...]
            m = jnp.max(logits, axis=1, keepdims=True)
            z = logits - m
            o_ref[i * BM:(i + 1) * BM, :] = z - jnp.log(
                jnp.sum(jnp.exp(z), axis=1, keepdims=True))


def kernel(x, adj, W1, b1, bn_gamma, bn_beta, bn_mean, bn_var, Wa, ba):
    nfeat = x.shape[1]
    nhid = W1.shape[1]
    nheads, _, nclass = Wa.shape
    ncat = nheads * nclass

    scale = (bn_gamma / jnp.sqrt(bn_var + 1e-5)).reshape(1, nhid)
    c = ((b1 - bn_mean) * scale[0] + bn_beta).reshape(1, nhid)
    wa_cat = jnp.transpose(Wa, (1, 0, 2)).reshape(nhid, ncat).astype(jnp.bfloat16)
    ba_cat = ba.reshape(1, ncat)

    t = pl.pallas_call(
        _prep_kernel,
        out_shape=jax.ShapeDtypeStruct((N, nhid), jnp.bfloat16),
    )(x.astype(jnp.bfloat16), W1.astype(jnp.bfloat16), scale)

    out = pl.pallas_call(
        _gcn_kernel,
        grid=(NB,),
        in_specs=[
            pl.BlockSpec((BM, N), lambda j: (j, 0)),
            pl.BlockSpec((N, nhid), lambda j: (0, 0)),
            pl.BlockSpec((1, nhid), lambda j: (0, 0)),
            pl.BlockSpec((nhid, ncat), lambda j: (0, 0)),
            pl.BlockSpec((1, ncat), lambda j: (0, 0)),
        ],
        out_specs=pl.BlockSpec((N, ncat), lambda j: (0, 0)),
        out_shape=jax.ShapeDtypeStruct((N, ncat), jnp.float32),
        scratch_shapes=[
            pltpu.VMEM((NB, N, BM), jnp.bfloat16),
            pltpu.VMEM((N, ncat), jnp.bfloat16),
            pltpu.VMEM((N, ncat), jnp.float32),
        ],
        compiler_params=pltpu.CompilerParams(
            vmem_limit_bytes=62 * 1024 * 1024),
    )(adj, t, c, wa_cat, ba_cat)
    return out


# speedup vs baseline: 1.3720x; 1.3720x over previous
import jax
import jax.numpy as jnp
from jax.experimental import pallas as pl
from jax.experimental.pallas import tpu as pltpu

N = 4096
BM = 512
NB = N // BM

F8 = jnp.float8_e4m3fn
ADJ_SCALE = 2.0 ** 14
P_SCALE = 2.0 ** 7
OUT_SCALE = 2.0 ** -21


def _prep_kernel(x_ref, w1_ref, scale_ref, t_ref):
    t = jnp.dot(x_ref[...], w1_ref[...], preferred_element_type=jnp.float32)
    t_ref[...] = (t * scale_ref[...]).astype(F8)


def _gcn_kernel(adj_ref, t_ref, c_ref, wa_ref, ba_ref,
                o_ref, adjv_ref, p_ref, acc_ref):
    j = pl.program_id(0)

    @pl.when(j == 0)
    def _zero():
        acc_ref[...] = jnp.zeros(acc_ref.shape, jnp.float32)

    ablk = (adj_ref[...] * ADJ_SCALE).astype(F8)
    for k in range(NB):
        adjv_ref[k, pl.ds(j * BM, BM), :] = ablk[:, k * BM:(k + 1) * BM]

    h = jnp.dot(ablk, t_ref[...], preferred_element_type=jnp.float32)
    h = jnp.maximum(h + c_ref[...], 0.0)
    h8 = (h * (P_SCALE / ADJ_SCALE)).astype(F8)
    pj = jnp.dot(h8, wa_ref[...],
                 preferred_element_type=jnp.float32).astype(F8)
    p_ref[pl.ds(j * BM, BM), :] = pj

    for k in range(NB):
        @pl.when(j > k)
        def _casea(k=k):
            blk = jnp.dot(ablk[:, k * BM:(k + 1) * BM],
                          p_ref[k * BM:(k + 1) * BM, :],
                          preferred_element_type=jnp.float32)
            acc_ref[pl.ds(j * BM, BM), :] += blk

    for i in range(NB):
        @pl.when(j >= i)
        def _caseb(i=i):
            blk = jnp.dot(adjv_ref[j, i * BM:(i + 1) * BM, :], pj,
                          preferred_element_type=jnp.float32)
            acc_ref[i * BM:(i + 1) * BM, :] += blk

    @pl.when(j == NB - 1)
    def _final():
        for i in range(NB):
            logits = acc_ref[i * BM:(i + 1) * BM, :] * OUT_SCALE + ba_ref[...]
            m = jnp.max(logits, axis=1, keepdims=True)
            z = logits - m
            o_ref[i * BM:(i + 1) * BM, :] = z - jnp.log(
                jnp.sum(jnp.exp(z), axis=1, keepdims=True))


def kernel(x, adj, W1, b1, bn_gamma, bn_beta, bn_mean, bn_var, Wa, ba):
    nfeat = x.shape[1]
    nhid = W1.shape[1]
    nheads, _, nclass = Wa.shape
    ncat = nheads * nclass

    scale = (bn_gamma / jnp.sqrt(bn_var + 1e-5)).reshape(1, nhid)
    c = (((b1 - bn_mean) * scale[0] + bn_beta) * ADJ_SCALE).reshape(1, nhid)
    wa_cat = jnp.transpose(Wa, (1, 0, 2)).reshape(nhid, ncat).astype(F8)
    ba_cat = ba.reshape(1, ncat)

    t = pl.pallas_call(
        _prep_kernel,
        out_shape=jax.ShapeDtypeStruct((N, nhid), F8),
    )(x.astype(jnp.bfloat16), W1.astype(jnp.bfloat16), scale)

    out = pl.pallas_call(
        _gcn_kernel,
        grid=(NB,),
        in_specs=[
            pl.BlockSpec((BM, N), lambda j: (j, 0)),
            pl.BlockSpec((N, nhid), lambda j: (0, 0)),
            pl.BlockSpec((1, nhid), lambda j: (0, 0)),
            pl.BlockSpec((nhid, ncat), lambda j: (0, 0)),
            pl.BlockSpec((1, ncat), lambda j: (0, 0)),
        ],
        out_specs=pl.BlockSpec((N, ncat), lambda j: (0, 0)),
        out_shape=jax.ShapeDtypeStruct((N, ncat), jnp.float32),
        scratch_shapes=[
            pltpu.VMEM((NB, N, BM), F8),
            pltpu.VMEM((N, ncat), F8),
            pltpu.VMEM((N, ncat), jnp.float32),
        ],
        compiler_params=pltpu.CompilerParams(
            vmem_limit_bytes=62 * 1024 * 1024),
    )(adj, t, c, wa_cat, ba_cat)
    return out


# speedup vs baseline: 1.3898x; 1.0130x over previous
import jax
import jax.numpy as jnp
from jax.experimental import pallas as pl
from jax.experimental.pallas import tpu as pltpu

N = 4096
BS = 256
NS = 2
NSTEPS = N // (BS * NS)
BM = 512
NB = N // BM

BF = jnp.bfloat16


def _prep_kernel(x_ref, w1_ref, scale_ref, t_ref):
    t = jnp.dot(x_ref[...], w1_ref[...], preferred_element_type=jnp.float32)
    t_ref[...] = (t * scale_ref[...]).astype(BF)


def _gcn_kernel(a0_ref, a1_ref, t_ref, c_ref, wa_ref, ba_ref,
                o_ref, adjv_ref, p_ref):
    j = pl.program_id(0)

    @pl.when(j < NSTEPS)
    def _stage0():
        for half, a_ref in ((0, a0_ref), (1, a1_ref)):
            rr = (NS * j + half) * BS
            ablk = a_ref[...].astype(BF)
            adjv_ref[pl.ds(rr, BS), :] = ablk
            h = jnp.dot(ablk, t_ref[...], preferred_element_type=jnp.float32)
            h = jnp.maximum(h + c_ref[...], 0.0)
            p_ref[pl.ds(rr, BS), :] = jnp.dot(
                h.astype(BF), wa_ref[...],
                preferred_element_type=jnp.float32).astype(BF)

    @pl.when(j == NSTEPS)
    def _stage1():
        for i in range(NB):
            logits = jnp.dot(adjv_ref[i * BM:(i + 1) * BM, :], p_ref[...],
                             preferred_element_type=jnp.float32)
            logits = logits + ba_ref[...]
            m = jnp.max(logits, axis=1, keepdims=True)
            z = logits - m
            o_ref[i * BM:(i + 1) * BM, :] = z - jnp.log(
                jnp.sum(jnp.exp(z), axis=1, keepdims=True))


def kernel(x, adj, W1, b1, bn_gamma, bn_beta, bn_mean, bn_var, Wa, ba):
    nfeat = x.shape[1]
    nhid = W1.shape[1]
    nheads, _, nclass = Wa.shape
    ncat = nheads * nclass

    scale = (bn_gamma / jnp.sqrt(bn_var + 1e-5)).reshape(1, nhid)
    c = ((b1 - bn_mean) * scale[0] + bn_beta).reshape(1, nhid)
    wa_cat = jnp.transpose(Wa, (1, 0, 2)).reshape(nhid, ncat).astype(BF)
    ba_cat = ba.reshape(1, ncat)

    t = pl.pallas_call(
        _prep_kernel,
        out_shape=jax.ShapeDtypeStruct((N, nhid), BF),
    )(x.astype(BF), W1.astype(BF), scale)

    nblk = N // BS
    out = pl.pallas_call(
        _gcn_kernel,
        grid=(NSTEPS + 1,),
        in_specs=[
            pl.BlockSpec((BS, N),
                         lambda j: (jnp.where(j < NSTEPS, NS * j, nblk - NS), 0)),
            pl.BlockSpec((BS, N),
                         lambda j: (jnp.where(j < NSTEPS, NS * j + 1, nblk - 1), 0)),
            pl.BlockSpec((N, nhid), lambda j: (0, 0)),
            pl.BlockSpec((1, nhid), lambda j: (0, 0)),
            pl.BlockSpec((nhid, ncat), lambda j: (0, 0)),
            pl.BlockSpec((1, ncat), lambda j: (0, 0)),
        ],
        out_specs=pl.BlockSpec((N, ncat), lambda j: (0, 0)),
        out_shape=jax.ShapeDtypeStruct((N, ncat), jnp.float32),
        scratch_shapes=[
            pltpu.VMEM((N, N), BF),
            pltpu.VMEM((N, ncat), BF),
        ],
        compiler_params=pltpu.CompilerParams(
            vmem_limit_bytes=62 * 1024 * 1024),
    )(adj, adj, t, c, wa_cat, ba_cat)
    return out
